# baseline (speedup 1.0000x reference)
import jax
import jax.numpy as jnp
from jax import lax
from jax.experimental import pallas as pl
from jax.experimental.pallas import tpu as pltpu
from jax.experimental.pallas import tpu_sc as plsc

BATCH = 16384
D_MODEL = 128
DIM = 512

_N_TILES = 32
_B_PER_W = BATCH // _N_TILES
_GATHER_WINDOW = 128
_NWIN = _B_PER_W // _GATHER_WINDOW

_vector_mesh = plsc.VectorSubcoreMesh(
    core_axis_name="core", subcore_axis_name="subcore"
)


@pl.kernel(
    out_type=jax.ShapeDtypeStruct((BATCH, D_MODEL), jnp.float32),
    mesh=_vector_mesh,
    scratch_types=[
        pltpu.VMEM((_NWIN, _GATHER_WINDOW), jnp.int32),
        pltpu.VMEM((_B_PER_W, D_MODEL), jnp.float32),
    ] + [pltpu.SemaphoreType.DMA] * (_NWIN + 1),
)
def _sc_gather_kernel(table_hbm, i_hbm, o_hbm, idx_v, rows_v,
                      g0, g1, g2, g3, wsem):
    wid = lax.axis_index("subcore") * 2 + lax.axis_index("core")
    base = wid * _B_PER_W
    gsems = (g0, g1, g2, g3)
    pltpu.sync_copy(i_hbm.at[wid], idx_v)
    copies = [
        pltpu.async_copy(
            table_hbm.at[idx_v.at[j]],
            rows_v.at[pl.ds(j * _GATHER_WINDOW, _GATHER_WINDOW)],
            gsems[j],
        )
        for j in range(_NWIN)
    ]
    writes = []
    for j in range(_NWIN):
        copies[j].wait()
        writes.append(pltpu.async_copy(
            rows_v.at[pl.ds(j * _GATHER_WINDOW, _GATHER_WINDOW)],
            o_hbm.at[pl.ds(base + j * _GATHER_WINDOW, _GATHER_WINDOW)],
            wsem,
        ))
    for w in writes:
        w.wait()


_MLP_BLK = 2048


def _mlp_body(emb_ref, w1_ref, b1_ref, w2_ref, b2_ref, out_ref):
    h = jnp.dot(emb_ref[...].astype(jnp.bfloat16), w1_ref[...],
                preferred_element_type=jnp.float32) + b1_ref[...]
    h = h * (0.5 + 0.5 * jnp.tanh(0.5 * h))
    out_ref[...] = jnp.dot(h.astype(jnp.bfloat16), w2_ref[...],
                           preferred_element_type=jnp.float32) + b2_ref[...]


_mlp = pl.pallas_call(
    _mlp_body,
    grid=(BATCH // _MLP_BLK,),
    in_specs=[
        pl.BlockSpec((_MLP_BLK, D_MODEL), lambda i: (i, 0)),
        pl.BlockSpec((D_MODEL, DIM), lambda i: (0, 0)),
        pl.BlockSpec((1, DIM), lambda i: (0, 0)),
        pl.BlockSpec((DIM, DIM), lambda i: (0, 0)),
        pl.BlockSpec((1, DIM), lambda i: (0, 0)),
    ],
    out_specs=pl.BlockSpec((_MLP_BLK, DIM), lambda i: (i, 0)),
    out_shape=jax.ShapeDtypeStruct((BATCH, DIM), jnp.float32),
    compiler_params=pltpu.CompilerParams(
        dimension_semantics=("parallel",)),
)


def kernel(t, table, W1, b1, W2, b2):
    idx = t.astype(jnp.int32).reshape(_N_TILES, _NWIN, _GATHER_WINDOW)
    emb = _sc_gather_kernel(table, idx)
    return _mlp(emb, W1.astype(jnp.bfloat16), b1.reshape(1, DIM),
                W2.astype(jnp.bfloat16), b2.reshape(1, DIM))


# speedup vs baseline: 1.0192x; 1.0192x over previous
import jax
import jax.numpy as jnp
from jax import lax
from jax.experimental import pallas as pl
from jax.experimental.pallas import tpu as pltpu
from jax.experimental.pallas import tpu_sc as plsc

BATCH = 16384
D_MODEL = 128
DIM = 512

_N_TILES = 32
_B_PER_W = BATCH // _N_TILES
_GATHER_WINDOW = 128
_NWIN = _B_PER_W // _GATHER_WINDOW

_vector_mesh = plsc.VectorSubcoreMesh(
    core_axis_name="core", subcore_axis_name="subcore"
)


@pl.kernel(
    out_type=jax.ShapeDtypeStruct((BATCH, D_MODEL), jnp.float32),
    mesh=_vector_mesh,
    scratch_types=[
        pltpu.VMEM((_NWIN, _GATHER_WINDOW), jnp.int32),
        pltpu.VMEM((_B_PER_W, D_MODEL), jnp.float32),
    ] + [pltpu.SemaphoreType.DMA] * (_NWIN + 1),
)
def _sc_gather_kernel(table_hbm, i_hbm, o_hbm, idx_v, rows_v,
                      g0, g1, g2, g3, wsem):
    wid = lax.axis_index("subcore") * 2 + lax.axis_index("core")
    base = wid * _B_PER_W
    gsems = (g0, g1, g2, g3)
    pltpu.sync_copy(i_hbm.at[wid], idx_v)
    copies = [
        pltpu.async_copy(
            table_hbm.at[idx_v.at[j]],
            rows_v.at[pl.ds(j * _GATHER_WINDOW, _GATHER_WINDOW)],
            gsems[j],
        )
        for j in range(_NWIN)
    ]
    writes = []
    for j in range(_NWIN):
        copies[j].wait()
        writes.append(pltpu.async_copy(
            rows_v.at[pl.ds(j * _GATHER_WINDOW, _GATHER_WINDOW)],
            o_hbm.at[pl.ds(base + j * _GATHER_WINDOW, _GATHER_WINDOW)],
            wsem,
        ))
    for w in writes:
        w.wait()


_MLP_BLK = 2048


def _mlp_body(emb_ref, w1_ref, b1_ref, w2_ref, b2_ref, out_ref):
    h = jnp.dot(emb_ref[...], w1_ref[...],
                preferred_element_type=jnp.float32) + b1_ref[...]
    h = h * (0.5 + 0.5 * jnp.tanh(0.5 * h))
    out_ref[...] = jnp.dot(h, w2_ref[...],
                           preferred_element_type=jnp.float32) + b2_ref[...]


_mlp = pl.pallas_call(
    _mlp_body,
    grid=(BATCH // _MLP_BLK,),
    in_specs=[
        pl.BlockSpec((_MLP_BLK, D_MODEL), lambda i: (i, 0)),
        pl.BlockSpec((D_MODEL, DIM), lambda i: (0, 0)),
        pl.BlockSpec((1, DIM), lambda i: (0, 0)),
        pl.BlockSpec((DIM, DIM), lambda i: (0, 0)),
        pl.BlockSpec((1, DIM), lambda i: (0, 0)),
    ],
    out_specs=pl.BlockSpec((_MLP_BLK, DIM), lambda i: (i, 0)),
    out_shape=jax.ShapeDtypeStruct((BATCH, DIM), jnp.float32),
    compiler_params=pltpu.CompilerParams(
        dimension_semantics=("parallel",)),
)


def kernel(t, table, W1, b1, W2, b2):
    idx = t.astype(jnp.int32).reshape(_N_TILES, _NWIN, _GATHER_WINDOW)
    emb = _sc_gather_kernel(table, idx)
    return _mlp(emb, W1, b1.reshape(1, DIM), W2, b2.reshape(1, DIM))


# speedup vs baseline: 1.0360x; 1.0165x over previous
import jax
import jax.numpy as jnp
from jax import lax
from jax.experimental import pallas as pl
from jax.experimental.pallas import tpu as pltpu
from jax.experimental.pallas import tpu_sc as plsc

BATCH = 16384
D_MODEL = 128
DIM = 512

_N_TILES = 32
_B_PER_W = BATCH // _N_TILES
_GATHER_WINDOW = 128
_NWIN = _B_PER_W // _GATHER_WINDOW

_vector_mesh = plsc.VectorSubcoreMesh(
    core_axis_name="core", subcore_axis_name="subcore"
)


@pl.kernel(
    out_type=jax.ShapeDtypeStruct((BATCH, D_MODEL), jnp.float32),
    mesh=_vector_mesh,
    scratch_types=[
        pltpu.VMEM((_NWIN, _GATHER_WINDOW), jnp.int32),
        pltpu.VMEM((_B_PER_W, D_MODEL), jnp.float32),
    ] + [pltpu.SemaphoreType.DMA] * (_NWIN + 1),
)
def _sc_gather_kernel(table_hbm, i_hbm, o_hbm, idx_v, rows_v,
                      g0, g1, g2, g3, wsem):
    wid = lax.axis_index("subcore") * 2 + lax.axis_index("core")
    base = wid * _B_PER_W
    gsems = (g0, g1, g2, g3)
    pltpu.sync_copy(i_hbm.at[wid], idx_v)
    copies = [
        pltpu.async_copy(
            table_hbm.at[idx_v.at[j]],
            rows_v.at[pl.ds(j * _GATHER_WINDOW, _GATHER_WINDOW)],
            gsems[j],
        )
        for j in range(_NWIN)
    ]
    writes = []
    for j in range(_NWIN):
        copies[j].wait()
        writes.append(pltpu.async_copy(
            rows_v.at[pl.ds(j * _GATHER_WINDOW, _GATHER_WINDOW)],
            o_hbm.at[pl.ds(base + j * _GATHER_WINDOW, _GATHER_WINDOW)],
            wsem,
        ))
    for w in writes:
        w.wait()


_MLP_BLK = 4096


def _mlp_body(emb_ref, w1_ref, b1_ref, w2_ref, b2_ref, out_ref):
    h = jnp.dot(emb_ref[...], w1_ref[...],
                preferred_element_type=jnp.float32) + b1_ref[...]
    h = h * (0.5 + 0.5 * jnp.tanh(0.5 * h))
    out_ref[...] = jnp.dot(h, w2_ref[...],
                           preferred_element_type=jnp.float32) + b2_ref[...]


_mlp = pl.pallas_call(
    _mlp_body,
    grid=(BATCH // _MLP_BLK,),
    in_specs=[
        pl.BlockSpec((_MLP_BLK, D_MODEL), lambda i: (i, 0)),
        pl.BlockSpec((D_MODEL, DIM), lambda i: (0, 0)),
        pl.BlockSpec((1, DIM), lambda i: (0, 0)),
        pl.BlockSpec((DIM, DIM), lambda i: (0, 0)),
        pl.BlockSpec((1, DIM), lambda i: (0, 0)),
    ],
    out_specs=pl.BlockSpec((_MLP_BLK, DIM), lambda i: (i, 0)),
    out_shape=jax.ShapeDtypeStruct((BATCH, DIM), jnp.float32),
    compiler_params=pltpu.CompilerParams(
        dimension_semantics=("parallel",)),
)


def kernel(t, table, W1, b1, W2, b2):
    idx = t.astype(jnp.int32).reshape(_N_TILES, _NWIN, _GATHER_WINDOW)
    emb = _sc_gather_kernel(table, idx)
    return _mlp(emb, W1, b1.reshape(1, DIM), W2, b2.reshape(1, DIM))
